# Initial kernel scaffold; baseline (speedup 1.0000x reference)
import jax
import jax.numpy as jnp
from jax.experimental import pallas as pl


def kernel(question1, question2, emb_table, W, b):
    raise NotImplementedError("write your pallas kernel here")


# speedup vs baseline: 3.7618x; 3.7618x over previous
import functools

import jax
import jax.numpy as jnp
from jax import lax
from jax.experimental import pallas as pl
from jax.experimental.pallas import tpu as pltpu
from jax.experimental.pallas import tpu_sc as plsc

SEQ = 50
BATCH = 1024
DIM = 64
ROWS_BLK = 4096
NUM_WORKERS = 32
COLS_PER_W = BATCH // NUM_WORKERS
LANES = 16


def _proj_body(tbl_ref, w_ref, b_ref, out_ref):
    x = tbl_ref[...]
    w = w_ref[...]
    out_ref[...] = jnp.sum(x * w, axis=1) + b_ref[0]


def _project_table(emb_table, W, b):
    vocab = emb_table.shape[0]
    grid = pl.cdiv(vocab, ROWS_BLK)
    vp = grid * ROWS_BLK
    return pl.pallas_call(
        _proj_body,
        grid=(grid,),
        in_specs=[
            pl.BlockSpec((ROWS_BLK, DIM), lambda i: (i, 0)),
            pl.BlockSpec((1, DIM), lambda i: (0, 0)),
            pl.BlockSpec(memory_space=pltpu.SMEM),
        ],
        out_specs=pl.BlockSpec((ROWS_BLK,), lambda i: (i,)),
        out_shape=jax.ShapeDtypeStruct((vp,), jnp.float32),
    )(emb_table, W, b)


def _rsqrt(x):
    m = x
    r = jnp.full_like(x, 1.0)
    for k in (32, 16, 8, 4, 2, 1):
        big = m >= 4.0 ** k
        m = jnp.where(big, m * 4.0 ** (-k), m)
        r = jnp.where(big, r * 2.0 ** (-k), r)
    for k in (32, 16, 8, 4, 2, 1):
        small = m < 4.0 ** (1 - k)
        m = jnp.where(small, m * 4.0 ** k, m)
        r = jnp.where(small, r * 2.0 ** k, r)
    y = 1.4 - 0.4583 * m + 0.0583 * m * m
    for _ in range(4):
        y = y * (1.5 - 0.5 * m * y * y)
    return y * r


IDX_PER_W = SEQ * COLS_PER_W


def _sc_cos_sim(p, q1r, q2r):
    mesh = plsc.VectorSubcoreMesh(core_axis_name="c", subcore_axis_name="s")

    @functools.partial(
        pl.kernel,
        mesh=mesh,
        out_type=jax.ShapeDtypeStruct((BATCH,), jnp.float32),
        scratch_types=[
            pltpu.VMEM((IDX_PER_W,), jnp.int32),
            pltpu.VMEM((IDX_PER_W,), jnp.int32),
            pltpu.VMEM((IDX_PER_W,), jnp.float32),
            pltpu.VMEM((IDX_PER_W,), jnp.float32),
            pltpu.VMEM((COLS_PER_W,), jnp.float32),
            pltpu.SemaphoreType.DMA,
            pltpu.SemaphoreType.DMA,
        ],
    )
    def k(p_hbm, q1_hbm, q2_hbm, out_hbm, idx1, idx2, v1, v2, ob, sem1, sem2):
        wid = lax.axis_index("s") * 2 + lax.axis_index("c")
        base = wid * COLS_PER_W
        pltpu.sync_copy(q1_hbm.at[wid], idx1)
        pltpu.sync_copy(q2_hbm.at[wid], idx2)
        c1 = pltpu.async_copy(p_hbm.at[idx1], v1, sem1)
        c2 = pltpu.async_copy(p_hbm.at[idx2], v2, sem2)
        c1.wait()
        c2.wait()
        for g in range(COLS_PER_W // LANES):
            def body(s, carry):
                d, a, c = carry
                x = v1[pl.ds(s * COLS_PER_W + g * LANES, LANES)]
                y = v2[pl.ds(s * COLS_PER_W + g * LANES, LANES)]
                return d + x * y, a + x * x, c + y * y
            z = jnp.zeros((LANES,), jnp.float32)
            d, a, c = lax.fori_loop(0, SEQ, body, (z, z, z))
            denom2 = jnp.maximum(a, 1e-16) * jnp.maximum(c, 1e-16)
            cos = d * _rsqrt(denom2)
            ob[pl.ds(g * LANES, LANES)] = 1.0 / (1.0 + jnp.exp(-cos))
        pltpu.sync_copy(ob, out_hbm.at[pl.ds(base, COLS_PER_W)])

    return k(p, q1r, q2r)


def _rearrange_idx(q):
    return (q.reshape(SEQ, NUM_WORKERS, COLS_PER_W)
             .transpose(1, 0, 2)
             .reshape(NUM_WORKERS, IDX_PER_W))


def kernel(question1, question2, emb_table, W, b):
    q1r = _rearrange_idx(question1.astype(jnp.int32))
    q2r = _rearrange_idx(question2.astype(jnp.int32))
    p = _project_table(emb_table, W, b)
    return _sc_cos_sim(p, q1r, q2r)


# speedup vs baseline: 4.4519x; 1.1835x over previous
import functools

import jax
import jax.numpy as jnp
import numpy as np
from jax import lax
from jax.experimental import pallas as pl
from jax.experimental.pallas import tpu as pltpu
from jax.experimental.pallas import tpu_sc as plsc

SEQ = 50
BATCH = 1024
DIM = 64
ROWS_BLK = 4096
LANES_TC = 128
NUM_WORKERS = 32
COLS_PER_W = BATCH // NUM_WORKERS
LANES = 16
IDX_PER_W = SEQ * COLS_PER_W

_ROW = np.arange(ROWS_BLK)
_DIAG = (np.equal.outer(_ROW % LANES_TC, np.arange(LANES_TC))
         .astype(np.float32))
_SEL = (np.equal.outer(np.arange(ROWS_BLK // LANES_TC), _ROW // LANES_TC)
        .astype(np.float32))


def _proj_body(tbl_ref, w_ref, diag_ref, sel_ref, b_ref, out_ref):
    x = tbl_ref[...]
    w = w_ref[...]
    t = jnp.dot(x, w, preferred_element_type=jnp.float32)
    tm = t * diag_ref[...]
    out_ref[...] = (
        jnp.dot(sel_ref[...], tm, preferred_element_type=jnp.float32)
        + b_ref[0]
    )


def _project_table(emb_table, W, b):
    vocab = emb_table.shape[0]
    grid = pl.cdiv(vocab, ROWS_BLK)
    w_rep = jnp.broadcast_to(W.reshape(DIM, 1), (DIM, LANES_TC))
    out_rows = grid * (ROWS_BLK // LANES_TC)
    p2d = pl.pallas_call(
        _proj_body,
        grid=(grid,),
        in_specs=[
            pl.BlockSpec((ROWS_BLK, DIM), lambda i: (i, 0)),
            pl.BlockSpec((DIM, LANES_TC), lambda i: (0, 0)),
            pl.BlockSpec((ROWS_BLK, LANES_TC), lambda i: (0, 0)),
            pl.BlockSpec((ROWS_BLK // LANES_TC, ROWS_BLK), lambda i: (0, 0)),
            pl.BlockSpec(memory_space=pltpu.SMEM),
        ],
        out_specs=pl.BlockSpec((ROWS_BLK // LANES_TC, LANES_TC),
                               lambda i: (i, 0)),
        out_shape=jax.ShapeDtypeStruct((out_rows, LANES_TC), jnp.float32),
    )(emb_table, w_rep, jnp.asarray(_DIAG), jnp.asarray(_SEL), b)
    return p2d.reshape(out_rows * LANES_TC)


def _rsqrt(x):
    m = x
    r = jnp.full_like(x, 1.0)
    for k in (32, 16, 8, 4, 2, 1):
        big = m >= 4.0 ** k
        m = jnp.where(big, m * 4.0 ** (-k), m)
        r = jnp.where(big, r * 2.0 ** (-k), r)
    for k in (32, 16, 8, 4, 2, 1):
        small = m < 4.0 ** (1 - k)
        m = jnp.where(small, m * 4.0 ** k, m)
        r = jnp.where(small, r * 2.0 ** k, r)
    y = 1.4 - 0.4583 * m + 0.0583 * m * m
    for _ in range(4):
        y = y * (1.5 - 0.5 * m * y * y)
    return y * r


def _sc_cos_sim(p, q1, q2):
    mesh = plsc.VectorSubcoreMesh(core_axis_name="c", subcore_axis_name="s")

    @functools.partial(
        pl.kernel,
        mesh=mesh,
        out_type=jax.ShapeDtypeStruct((BATCH,), jnp.float32),
        scratch_types=[
            pltpu.VMEM((IDX_PER_W,), jnp.int32),
            pltpu.VMEM((IDX_PER_W,), jnp.int32),
            pltpu.VMEM((IDX_PER_W,), jnp.float32),
            pltpu.VMEM((IDX_PER_W,), jnp.float32),
            pltpu.VMEM((COLS_PER_W,), jnp.float32),
            pltpu.SemaphoreType.DMA,
            pltpu.SemaphoreType.DMA,
        ],
    )
    def k(p_hbm, q1_hbm, q2_hbm, out_hbm, idx1, idx2, v1, v2, ob, sem1, sem2):
        wid = lax.axis_index("s") * 2 + lax.axis_index("c")
        base = wid * COLS_PER_W
        pltpu.sync_copy(q1_hbm.at[wid], idx1)
        pltpu.sync_copy(q2_hbm.at[wid], idx2)
        c1 = pltpu.async_copy(p_hbm.at[idx1], v1, sem1)
        c2 = pltpu.async_copy(p_hbm.at[idx2], v2, sem2)
        c1.wait()
        c2.wait()
        for g in range(COLS_PER_W // LANES):
            def body(s, carry):
                d, a, c = carry
                x = v1[pl.ds(s * COLS_PER_W + g * LANES, LANES)]
                y = v2[pl.ds(s * COLS_PER_W + g * LANES, LANES)]
                return d + x * y, a + x * x, c + y * y
            z = jnp.zeros((LANES,), jnp.float32)
            d, a, c = lax.fori_loop(0, SEQ, body, (z, z, z))
            denom2 = jnp.maximum(a, 1e-16) * jnp.maximum(c, 1e-16)
            cos = d * _rsqrt(denom2)
            ob[pl.ds(g * LANES, LANES)] = 1.0 / (1.0 + jnp.exp(-cos))
        pltpu.sync_copy(ob, out_hbm.at[pl.ds(base, COLS_PER_W)])

    return k(p, q1, q2)


def _rearrange_idx(q):
    return (q.reshape(SEQ, NUM_WORKERS, COLS_PER_W)
             .transpose(1, 0, 2)
             .reshape(NUM_WORKERS, IDX_PER_W))


def kernel(question1, question2, emb_table, W, b):
    q1r = _rearrange_idx(question1.astype(jnp.int32))
    q2r = _rearrange_idx(question2.astype(jnp.int32))
    p = _project_table(emb_table, W, b)
    return _sc_cos_sim(p, q1r, q2r)


# speedup vs baseline: 4.5642x; 1.0252x over previous
import functools

import jax
import jax.numpy as jnp
import numpy as np
from jax import lax
from jax.experimental import pallas as pl
from jax.experimental.pallas import tpu as pltpu
from jax.experimental.pallas import tpu_sc as plsc

SEQ = 50
BATCH = 1024
DIM = 64
ROWS_BLK = 4096
LANES_TC = 128
NUM_WORKERS = 32
COLS_PER_W = BATCH // NUM_WORKERS
LANES = 16
IDX_PER_W = SEQ * COLS_PER_W

_ROW = np.arange(ROWS_BLK)
_DIAG = (np.equal.outer(_ROW % LANES_TC, np.arange(LANES_TC))
         .astype(np.float32))
_SEL = (np.equal.outer(np.arange(ROWS_BLK // LANES_TC), _ROW // LANES_TC)
        .astype(np.float32))


def _proj_body(tbl_ref, w_ref, diag_ref, sel_ref, b_ref, out_ref):
    x = tbl_ref[...]
    w = w_ref[...]
    t = jnp.dot(x, w, preferred_element_type=jnp.float32)
    tm = t * diag_ref[...]
    out_ref[...] = (
        jnp.dot(sel_ref[...], tm, preferred_element_type=jnp.float32)
        + b_ref[0]
    )


def _project_table(emb_table, W, b):
    vocab = emb_table.shape[0]
    grid = pl.cdiv(vocab, ROWS_BLK)
    w_rep = jnp.broadcast_to(W.reshape(DIM, 1), (DIM, LANES_TC))
    out_rows = grid * (ROWS_BLK // LANES_TC)
    p2d = pl.pallas_call(
        _proj_body,
        grid=(grid,),
        in_specs=[
            pl.BlockSpec((ROWS_BLK, DIM), lambda i: (i, 0)),
            pl.BlockSpec((DIM, LANES_TC), lambda i: (0, 0)),
            pl.BlockSpec((ROWS_BLK, LANES_TC), lambda i: (0, 0)),
            pl.BlockSpec((ROWS_BLK // LANES_TC, ROWS_BLK), lambda i: (0, 0)),
            pl.BlockSpec(memory_space=pltpu.SMEM),
        ],
        out_specs=pl.BlockSpec((ROWS_BLK // LANES_TC, LANES_TC),
                               lambda i: (i, 0)),
        out_shape=jax.ShapeDtypeStruct((out_rows, LANES_TC), jnp.float32),
    )(emb_table, w_rep, jnp.asarray(_DIAG), jnp.asarray(_SEL), b)
    return p2d.reshape(out_rows * LANES_TC)


def _rsqrt(x):
    m = x
    r = jnp.full_like(x, 1.0)
    for k in (32, 16, 8, 4, 2, 1):
        big = m >= 4.0 ** k
        m = jnp.where(big, m * 4.0 ** (-k), m)
        r = jnp.where(big, r * 2.0 ** (-k), r)
    for k in (32, 16, 8, 4, 2, 1):
        small = m < 4.0 ** (1 - k)
        m = jnp.where(small, m * 4.0 ** k, m)
        r = jnp.where(small, r * 2.0 ** k, r)
    y = 1.4 - 0.4583 * m + 0.0583 * m * m
    for _ in range(4):
        y = y * (1.5 - 0.5 * m * y * y)
    return y * r


def _sc_cos_sim(p, q1, q2):
    mesh = plsc.VectorSubcoreMesh(core_axis_name="c", subcore_axis_name="s")

    @functools.partial(
        pl.kernel,
        mesh=mesh,
        out_type=jax.ShapeDtypeStruct((BATCH,), jnp.float32),
        scratch_types=[
            pltpu.VMEM((SEQ, 4 * COLS_PER_W), jnp.int32),
            pltpu.VMEM((SEQ, 4 * COLS_PER_W), jnp.int32),
            pltpu.VMEM((SEQ, COLS_PER_W), jnp.float32),
            pltpu.VMEM((SEQ, COLS_PER_W), jnp.float32),
            pltpu.VMEM((2 * IDX_PER_W,), jnp.float32),
            pltpu.VMEM((COLS_PER_W,), jnp.float32),
            pltpu.SemaphoreType.DMA,
        ],
    )
    def k(p_hbm, q1_hbm, q2_hbm, out_hbm, idx1, idx2, v1, v2, drain, ob, sem):
        wid = lax.axis_index("s") * 2 + lax.axis_index("c")
        base = wid * COLS_PER_W
        grp = (wid // 4) * (4 * COLS_PER_W)
        sub = (wid % 4) * COLS_PER_W
        pltpu.sync_copy(q1_hbm.at[:, pl.ds(grp, 4 * COLS_PER_W)], idx1)
        pltpu.sync_copy(q2_hbm.at[:, pl.ds(grp, 4 * COLS_PER_W)], idx2)
        def issue(s):
            pltpu.async_copy(p_hbm.at[idx1.at[s, pl.ds(sub, COLS_PER_W)]],
                             v1.at[s], sem)
            pltpu.async_copy(p_hbm.at[idx2.at[s, pl.ds(sub, COLS_PER_W)]],
                             v2.at[s], sem)
        pl.loop(0, SEQ, unroll=10)(issue)
        pltpu.make_async_copy(p_hbm.at[pl.ds(0, 2 * IDX_PER_W)], drain,
                              sem).wait()
        for g in range(COLS_PER_W // LANES):
            def body(s, carry):
                d, a, c = carry
                x = v1[s, pl.ds(g * LANES, LANES)]
                y = v2[s, pl.ds(g * LANES, LANES)]
                return d + x * y, a + x * x, c + y * y
            z = jnp.zeros((LANES,), jnp.float32)
            d, a, c = lax.fori_loop(0, SEQ, body, (z, z, z))
            denom2 = jnp.maximum(a, 1e-16) * jnp.maximum(c, 1e-16)
            cos = d * _rsqrt(denom2)
            ob[pl.ds(g * LANES, LANES)] = 1.0 / (1.0 + jnp.exp(-cos))
        pltpu.sync_copy(ob, out_hbm.at[pl.ds(base, COLS_PER_W)])

    return k(p, q1, q2)


def kernel(question1, question2, emb_table, W, b):
    q1 = question1.astype(jnp.int32)
    q2 = question2.astype(jnp.int32)
    p = _project_table(emb_table, W, b)
    return _sc_cos_sim(p, q1, q2)


# speedup vs baseline: 12.6590x; 2.7736x over previous
import functools

import jax
import jax.numpy as jnp
from jax import lax
from jax.experimental import pallas as pl
from jax.experimental.pallas import tpu as pltpu
from jax.experimental.pallas import tpu_sc as plsc

SEQ = 50
BATCH = 1024
DIM = 64
ROWS_BLK = 25600
LANES_TC = 128
NUM_WORKERS = 32
COLS_PER_W = BATCH // NUM_WORKERS
LANES = 16
IDX_PER_W = SEQ * COLS_PER_W

def _proj_body(tblt_ref, w_ref, b_ref, out_ref):
    x = tblt_ref[...]
    w = w_ref[...]
    t = jnp.dot(w, x, preferred_element_type=jnp.float32)
    out_ref[...] = t[0] + b_ref[0]


def _project_table(emb_table, W, b):
    tbl_t = emb_table.T
    vocab = emb_table.shape[0]
    grid = pl.cdiv(vocab, ROWS_BLK)
    return pl.pallas_call(
        _proj_body,
        grid=(grid,),
        in_specs=[
            pl.BlockSpec((DIM, ROWS_BLK), lambda i: (0, i)),
            pl.BlockSpec((1, DIM), lambda i: (0, 0)),
            pl.BlockSpec(memory_space=pltpu.SMEM),
        ],
        out_specs=pl.BlockSpec((ROWS_BLK,), lambda i: (i,)),
        out_shape=jax.ShapeDtypeStruct((grid * ROWS_BLK,), jnp.float32),
    )(tbl_t, W, b)


def _rsqrt(x):
    m = x
    r = jnp.full_like(x, 1.0)
    for k in (32, 16, 8, 4, 2, 1):
        big = m >= 4.0 ** k
        m = jnp.where(big, m * 4.0 ** (-k), m)
        r = jnp.where(big, r * 2.0 ** (-k), r)
    for k in (32, 16, 8, 4, 2, 1):
        small = m < 4.0 ** (1 - k)
        m = jnp.where(small, m * 4.0 ** k, m)
        r = jnp.where(small, r * 2.0 ** k, r)
    y = 1.4 - 0.4583 * m + 0.0583 * m * m
    for _ in range(4):
        y = y * (1.5 - 0.5 * m * y * y)
    return y * r


def _sc_cos_sim(p, q1, q2):
    mesh = plsc.VectorSubcoreMesh(core_axis_name="c", subcore_axis_name="s")

    @functools.partial(
        pl.kernel,
        mesh=mesh,
        out_type=jax.ShapeDtypeStruct((BATCH,), jnp.float32),
        scratch_types=[
            pltpu.VMEM((SEQ, 4 * COLS_PER_W), jnp.int32),
            pltpu.VMEM((SEQ, 4 * COLS_PER_W), jnp.int32),
            pltpu.VMEM((SEQ, COLS_PER_W), jnp.float32),
            pltpu.VMEM((SEQ, COLS_PER_W), jnp.float32),
            pltpu.VMEM((2 * IDX_PER_W,), jnp.float32),
            pltpu.VMEM((COLS_PER_W,), jnp.float32),
            pltpu.SemaphoreType.DMA,
            pltpu.SemaphoreType.DMA,
        ],
    )
    def k(p_hbm, q1_hbm, q2_hbm, out_hbm, idx1, idx2, v1, v2, drain, ob,
          sem, sem2):
        wid = lax.axis_index("s") * 2 + lax.axis_index("c")
        base = wid * COLS_PER_W
        grp = (wid // 4) * (4 * COLS_PER_W)
        sub = (wid % 4) * COLS_PER_W
        ca = pltpu.async_copy(q1_hbm.at[:, pl.ds(grp, 4 * COLS_PER_W)],
                              idx1, sem2)
        cb = pltpu.async_copy(q2_hbm.at[:, pl.ds(grp, 4 * COLS_PER_W)],
                              idx2, sem2)
        ca.wait()
        cb.wait()
        def issue(s):
            pltpu.async_copy(p_hbm.at[idx1.at[s, pl.ds(sub, COLS_PER_W)]],
                             v1.at[s], sem)
            pltpu.async_copy(p_hbm.at[idx2.at[s, pl.ds(sub, COLS_PER_W)]],
                             v2.at[s], sem)
        pl.loop(0, SEQ, unroll=10)(issue)
        pltpu.make_async_copy(p_hbm.at[pl.ds(0, 2 * IDX_PER_W)], drain,
                              sem).wait()
        for g in range(COLS_PER_W // LANES):
            def body(s, carry):
                d, a, c = carry
                x = v1[s, pl.ds(g * LANES, LANES)]
                y = v2[s, pl.ds(g * LANES, LANES)]
                return d + x * y, a + x * x, c + y * y
            z = jnp.zeros((LANES,), jnp.float32)
            d, a, c = lax.fori_loop(0, SEQ, body, (z, z, z))
            denom2 = jnp.maximum(a, 1e-16) * jnp.maximum(c, 1e-16)
            cos = d * _rsqrt(denom2)
            ob[pl.ds(g * LANES, LANES)] = 1.0 / (1.0 + jnp.exp(-cos))
        pltpu.sync_copy(ob, out_hbm.at[pl.ds(base, COLS_PER_W)])

    return k(p, q1, q2)


def kernel(question1, question2, emb_table, W, b):
    q1 = question1.astype(jnp.int32)
    q2 = question2.astype(jnp.int32)
    p = _project_table(emb_table, W, b)
    return _sc_cos_sim(p, q1, q2)
